# Initial kernel scaffold; baseline (speedup 1.0000x reference)
import jax
import jax.numpy as jnp
from jax.experimental import pallas as pl


def kernel(coordinates, edge_src, edge_dst):
    raise NotImplementedError("write your pallas kernel here")


# speedup vs baseline: 8.5893x; 8.5893x over previous
import functools
import math

import jax
import jax.numpy as jnp
from jax import lax
from jax.experimental import pallas as pl
from jax.experimental.pallas import tpu as pltpu
from jax.experimental.pallas import tpu_sc as plsc

CUTOFF = 5.0
NW = 32
LANES = 16
BLK = 2000

_MAGIC = 0x5F3759DF
_C3 = -1.0 / 6.0
_C5 = 1.0 / 120.0
_C7 = -1.0 / 5040.0
_C9 = 1.0 / 362880.0
_C11 = -1.0 / 39916800.0


def _rsqrt(d2):
    yi = lax.bitcast_convert_type(d2, jnp.int32)
    yi = jnp.int32(_MAGIC) - lax.shift_right_arithmetic(yi, jnp.int32(1))
    y = lax.bitcast_convert_type(yi, jnp.float32)
    for _ in range(3):
        y = y * (1.5 - 0.5 * d2 * y * y)
    return y


def _make_sc_call(n_edges):
    assert n_edges % (NW * BLK) == 0, n_edges
    epw = n_edges // NW
    nblk = epw // BLK
    groups = BLK // LANES

    mesh = plsc.VectorSubcoreMesh(core_axis_name="c", subcore_axis_name="s")

    @functools.partial(
        pl.kernel,
        mesh=mesh,
        compiler_params=pltpu.CompilerParams(needs_layout_passes=False),
        out_type=[
            jax.ShapeDtypeStruct((3 * n_edges,), jnp.float32),
            jax.ShapeDtypeStruct((n_edges,), jnp.float32),
            jax.ShapeDtypeStruct((n_edges,), jnp.float32),
            jax.ShapeDtypeStruct((n_edges,), jnp.int32),
        ],
        scratch_types=[
            pltpu.VMEM((BLK,), jnp.int32),
            pltpu.VMEM((BLK,), jnp.int32),
            pltpu.VMEM((BLK,), jnp.float32),
            pltpu.VMEM((BLK,), jnp.float32),
            pltpu.VMEM((BLK,), jnp.float32),
            pltpu.VMEM((BLK,), jnp.float32),
            pltpu.VMEM((BLK,), jnp.float32),
            pltpu.VMEM((BLK,), jnp.float32),
            pltpu.VMEM((3 * BLK,), jnp.float32),
            pltpu.VMEM((BLK,), jnp.float32),
            pltpu.VMEM((BLK,), jnp.float32),
            pltpu.VMEM((BLK,), jnp.int32),
            pltpu.SemaphoreType.DMA,
        ],
    )
    def sc_call(cx_hbm, cy_hbm, cz_hbm, src_hbm, dst_hbm,
                vec_hbm, dist_hbm, sw_hbm, mask_hbm,
                src_i, dst_i, sxv, syv, szv, dxv, dyv, dzv,
                vec_v, dist_v, sw_v, mask_v, sem):
        wid = lax.axis_index("s") * 2 + lax.axis_index("c")
        base = wid * epw

        iota = lax.iota(jnp.int32, LANES)
        zero_i = jnp.zeros((LANES,), jnp.int32)
        one_i = jnp.ones((LANES,), jnp.int32)
        two_i = one_i + one_i

        def block(b, carry):
            off = base + b * BLK
            pltpu.sync_copy(src_hbm.at[pl.ds(off, BLK)], src_i)
            pltpu.sync_copy(dst_hbm.at[pl.ds(off, BLK)], dst_i)
            cps = [
                pltpu.async_copy(cx_hbm.at[src_i], sxv, sem),
                pltpu.async_copy(cy_hbm.at[src_i], syv, sem),
                pltpu.async_copy(cz_hbm.at[src_i], szv, sem),
                pltpu.async_copy(cx_hbm.at[dst_i], dxv, sem),
                pltpu.async_copy(cy_hbm.at[dst_i], dyv, sem),
                pltpu.async_copy(cz_hbm.at[dst_i], dzv, sem),
            ]
            for cp in cps:
                cp.wait()

            def group(g, c2):
                sl = pl.ds(g * LANES, LANES)
                vx = dxv[sl] - sxv[sl]
                vy = dyv[sl] - syv[sl]
                vz = dzv[sl] - szv[sl]
                e3 = (g * LANES + iota) * 3
                plsc.store_scatter(vec_v, [e3], vx)
                plsc.store_scatter(vec_v, [e3 + one_i], vy)
                plsc.store_scatter(vec_v, [e3 + two_i], vz)
                d2 = vx * vx + vy * vy + vz * vz
                dist = d2 * _rsqrt(d2)
                mask = dist < CUTOFF
                u = dist * (1.0 / CUTOFF) - 0.5
                t = u * math.pi
                t2 = t * t
                p = jnp.float32(_C11)
                p = p * t2 + _C9
                p = p * t2 + _C7
                p = p * t2 + _C5
                p = p * t2 + _C3
                p = p * t2 + 1.0
                sn = t * p
                sw = jnp.where(mask, 0.5 - 0.5 * sn, 0.0)
                dist_v[sl] = dist
                sw_v[sl] = sw
                mask_v[sl] = jnp.where(mask, one_i, zero_i)
                return c2

            lax.fori_loop(0, groups, group, 0)

            pltpu.sync_copy(vec_v, vec_hbm.at[pl.ds(3 * off, 3 * BLK)])
            pltpu.sync_copy(dist_v, dist_hbm.at[pl.ds(off, BLK)])
            pltpu.sync_copy(sw_v, sw_hbm.at[pl.ds(off, BLK)])
            pltpu.sync_copy(mask_v, mask_hbm.at[pl.ds(off, BLK)])
            return carry

        lax.fori_loop(0, nblk, block, 0)

    return sc_call


@jax.jit
def kernel(coordinates, edge_src, edge_dst):
    n_edges = edge_src.shape[0]
    cx = coordinates[:, 0]
    cy = coordinates[:, 1]
    cz = coordinates[:, 2]
    vec_flat, dist, sw, mask = _make_sc_call(n_edges)(
        cx, cy, cz, edge_src, edge_dst)
    return (vec_flat.reshape(n_edges, 3), dist, sw, mask.astype(bool))
